# baseline (speedup 1.0000x reference)
import jax
import jax.numpy as jnp
from jax import lax
from jax.experimental import pallas as pl
from jax.experimental.pallas import tpu as pltpu
from jax.experimental.pallas import tpu_sc as plsc

B, L, H = 4, 4096, 1024
V, P, T = 30522, 4096, 2

NC, NS = 2, 16
NW = NC * NS
PPW = L // NW
CP = 8
NCH = PPW // CP
CT = B * CP
TB = 8
HC = H // 16


def _rsqrt16(x):
    i = plsc.bitcast(x, jnp.int32)
    i = 0x5F3759DF - lax.shift_right_logical(i, 1)
    y = plsc.bitcast(i, jnp.float32)
    xh = x * 0.5
    for _ in range(3):
        y = y * (1.5 - xh * y * y)
    return y


def _lanesum(x):
    lanes = lax.iota(jnp.int32, 16)
    for off in (8, 4, 2, 1):
        perm = lax.bitwise_and(lanes + off, 15)
        x = x + x.at[perm].get(mode="promise_in_bounds")
    return x


def _body(ids_hbm, ttf_hbm, word_hbm, pos_hbm, type_hbm, gamma_hbm, beta_hbm,
          out_hbm,
          idx_v, ttf_v, wbuf, posbuf, tybuf, tyd, gv, bv, gsem, psem, osem):
    wid = lax.axis_index("s") * NC + lax.axis_index("c")
    pwb = wid * PPW

    for b in range(B):
        pltpu.sync_copy(ids_hbm.at[pl.ds(b * L + pwb, PPW)],
                        idx_v.at[pl.ds(b * PPW, PPW)])
        pltpu.sync_copy(ttf_hbm.at[pl.ds(b * L + pwb, PPW)],
                        ttf_v.at[pl.ds(b * PPW, PPW)])
    pltpu.sync_copy(type_hbm, tybuf)
    pltpu.sync_copy(gamma_hbm, gv)
    pltpu.sync_copy(beta_hbm, bv)

    def tyd_body(k):
        sl = pl.ds(k * 16, 16)
        tyd[sl] = tybuf[1, sl] - tybuf[0, sl]

    plsc.parallel_loop(0, HC, unroll=2)(tyd_body)

    def word_copies(c, slot):
        coff = pl.multiple_of(c * CP, CP)
        return [pltpu.make_async_copy(
                    word_hbm.at[idx_v.at[pl.ds(b * PPW + coff, CP)]],
                    wbuf.at[slot, pl.ds(b * TB, TB)], gsem.at[slot])
                for b in range(B)]

    def out_copies(c, slot):
        coff = pl.multiple_of(c * CP, CP)
        return [pltpu.make_async_copy(
                    wbuf.at[slot, pl.ds(b * TB, TB)],
                    out_hbm.at[pl.ds(b * L + pwb + coff, CP)],
                    osem.at[slot])
                for b in range(B)]

    def start_chunk(c, slot):
        for cp in word_copies(c, slot):
            cp.start()
        pltpu.async_copy(pos_hbm.at[pl.ds(pwb + pl.multiple_of(c * CP, CP),
                                          CP)],
                         posbuf.at[slot], psem.at[slot])

    def do_chunk(c, slot):
        @pl.when(c + 1 < NCH)
        def _():
            @pl.when(c >= 1)
            def _():
                for cp in out_copies(c - 1, 1 - slot):
                    cp.wait()

            start_chunk(c + 1, 1 - slot)

        for cp in word_copies(c, slot):
            cp.wait()
        pltpu.make_async_copy(pos_hbm.at[pl.ds(pwb + pl.multiple_of(c * CP,
                                                                    CP), CP)],
                              posbuf.at[slot], psem.at[slot]).wait()

        z = jnp.zeros((16,), jnp.float32)
        for b in range(B):
            t0 = b * TB
            ttc = ttf_v[pl.ds(b * PPW + c * CP, 16)]
            fis = [ttc.at[jnp.full((16,), i, jnp.int32)].get(
                       mode="promise_in_bounds") for i in range(TB)]

            def acc_body(k, carry, t0=t0, fis=fis):
                s = list(carry[0:TB])
                q = list(carry[TB:2 * TB])
                sl = pl.ds(k * 16, 16)
                ty0 = tybuf[0, sl]
                tydv = tyd[sl]
                for i in range(TB):
                    v = (wbuf[slot, t0 + i, sl] + posbuf[slot, i, sl]
                         + (ty0 + fis[i] * tydv))
                    wbuf[slot, t0 + i, sl] = v
                    s[i] = s[i] + v
                    q[i] = q[i] + v * v
                return tuple(s) + tuple(q)

            carry = plsc.parallel_loop(
                0, HC, unroll=1, carry=(z,) * (2 * TB))(acc_body)
            ms, rs = [], []
            for i in range(TB):
                meanv = _lanesum(carry[i]) * (1.0 / H)
                varv = _lanesum(carry[TB + i]) * (1.0 / H) - meanv * meanv
                ms.append(meanv)
                rs.append(_rsqrt16(varv + 1e-12))

            def norm_body(k, t0=t0, ms=ms, rs=rs):
                sl = pl.ds(k * 16, 16)
                g = gv[sl]
                b_ = bv[sl]
                for i in range(TB):
                    v = wbuf[slot, t0 + i, sl]
                    wbuf[slot, t0 + i, sl] = (v - ms[i]) * rs[i] * g + b_

            plsc.parallel_loop(0, HC, unroll=2)(norm_body)

        for cp in out_copies(c, slot):
            cp.start()

    start_chunk(0, 0)

    def pair(cc, _):
        do_chunk(2 * cc, 0)
        do_chunk(2 * cc + 1, 1)
        return 0

    lax.fori_loop(0, NCH // 2, pair, 0)
    for j in range(2):
        for cp in out_copies(NCH - 2 + j, j):
            cp.wait()


@jax.jit
def _run(ids_flat, ttf_flat, word_emb, pos_emb, type_emb, gamma, beta):
    mesh = plsc.VectorSubcoreMesh(core_axis_name="c", subcore_axis_name="s")
    f = pl.kernel(
        _body,
        out_type=jax.ShapeDtypeStruct((B * L, H), jnp.float32),
        mesh=mesh,
        compiler_params=pltpu.CompilerParams(needs_layout_passes=False),
        scratch_types=[
            pltpu.VMEM((B * PPW,), jnp.int32),
            pltpu.VMEM((B * PPW + 16,), jnp.float32),
            pltpu.VMEM((2, CT, H), jnp.float32),
            pltpu.VMEM((2, CP, H), jnp.float32),
            pltpu.VMEM((T, H), jnp.float32),
            pltpu.VMEM((H,), jnp.float32),
            pltpu.VMEM((H,), jnp.float32),
            pltpu.VMEM((H,), jnp.float32),
            pltpu.SemaphoreType.DMA((2,)),
            pltpu.SemaphoreType.DMA((2,)),
            pltpu.SemaphoreType.DMA((2,)),
        ],
    )
    return f(ids_flat, ttf_flat, word_emb, pos_emb, type_emb, gamma, beta)


def kernel(input_ids, token_type_ids, word_emb, pos_emb, type_emb, gamma, beta):
    ids_flat = input_ids.reshape(-1).astype(jnp.int32)
    ttf_flat = token_type_ids.reshape(-1).astype(jnp.float32)
    out = _run(ids_flat, ttf_flat, word_emb, pos_emb, type_emb, gamma, beta)
    return out.reshape(B, L, H)


# speedup vs baseline: 1.0245x; 1.0245x over previous
import jax
import jax.numpy as jnp
from jax import lax
from jax.experimental import pallas as pl
from jax.experimental.pallas import tpu as pltpu
from jax.experimental.pallas import tpu_sc as plsc

B, L, H = 4, 4096, 1024
V, P, T = 30522, 4096, 2

NC, NS = 2, 16
NW = NC * NS
NBP = B // 2
WPP = NW // NBP
PPW = L // WPP
CP = 16
NCH = PPW // CP
CT = 2 * CP
TB = 8
HC = H // 16


def _rsqrt16(x):
    i = plsc.bitcast(x, jnp.int32)
    i = 0x5F3759DF - lax.shift_right_logical(i, 1)
    y = plsc.bitcast(i, jnp.float32)
    xh = x * 0.5
    for _ in range(3):
        y = y * (1.5 - xh * y * y)
    return y


def _lanesum(x):
    lanes = lax.iota(jnp.int32, 16)
    for off in (8, 4, 2, 1):
        perm = lax.bitwise_and(lanes + off, 15)
        x = x + x.at[perm].get(mode="promise_in_bounds")
    return x


def _body(ids_hbm, ttf_hbm, word_hbm, pos_hbm, type_hbm, gamma_hbm, beta_hbm,
          out_hbm,
          idx_v, ttf_v, wbuf, posbuf, tybuf, tyd, gv, bv, gsem, psem, osem):
    wid = lax.axis_index("s") * NC + lax.axis_index("c")
    g = wid // WPP
    pwb = lax.rem(wid, WPP) * PPW
    b0 = g * 2

    for jb in range(2):
        pltpu.sync_copy(ids_hbm.at[pl.ds((b0 + jb) * L + pwb, PPW)],
                        idx_v.at[pl.ds(jb * PPW, PPW)])
        pltpu.sync_copy(ttf_hbm.at[pl.ds((b0 + jb) * L + pwb, PPW)],
                        ttf_v.at[pl.ds(jb * PPW, PPW)])
    pltpu.sync_copy(type_hbm, tybuf)
    pltpu.sync_copy(gamma_hbm, gv)
    pltpu.sync_copy(beta_hbm, bv)

    def tyd_body(k):
        sl = pl.ds(k * 16, 16)
        tyd[sl] = tybuf[1, sl] - tybuf[0, sl]

    plsc.parallel_loop(0, HC, unroll=2)(tyd_body)

    def word_copies(c, slot):
        coff = pl.multiple_of(c * CP, CP)
        return [pltpu.make_async_copy(
                    word_hbm.at[idx_v.at[pl.ds(jb * PPW + coff, CP)]],
                    wbuf.at[slot, pl.ds(jb * CP, CP)], gsem.at[slot])
                for jb in range(2)]

    def out_copies(c, slot):
        coff = pl.multiple_of(c * CP, CP)
        return [pltpu.make_async_copy(
                    wbuf.at[slot, pl.ds(jb * CP, CP)],
                    out_hbm.at[pl.ds((b0 + jb) * L + pwb + coff, CP)],
                    osem.at[slot])
                for jb in range(2)]

    def start_chunk(c, slot):
        for cp in word_copies(c, slot):
            cp.start()
        pltpu.async_copy(pos_hbm.at[pl.ds(pwb + pl.multiple_of(c * CP, CP),
                                          CP)],
                         posbuf.at[slot], psem.at[slot])

    def do_chunk(c, slot):
        @pl.when(c + 1 < NCH)
        def _():
            @pl.when(c >= 1)
            def _():
                for cp in out_copies(c - 1, 1 - slot):
                    cp.wait()

            start_chunk(c + 1, 1 - slot)

        for cp in word_copies(c, slot):
            cp.wait()
        pltpu.make_async_copy(pos_hbm.at[pl.ds(pwb + pl.multiple_of(c * CP,
                                                                    CP), CP)],
                              posbuf.at[slot], psem.at[slot]).wait()

        z = jnp.zeros((16,), jnp.float32)
        for j in range(CT // TB):
            t0 = j * TB
            p0 = (j % 2) * TB
            ttc = ttf_v[pl.ds((j // 2) * PPW + c * CP + p0, 16)]
            fis = [ttc.at[jnp.full((16,), i, jnp.int32)].get(
                       mode="promise_in_bounds") for i in range(TB)]

            def acc_body(k, carry, t0=t0, p0=p0, fis=fis):
                s = list(carry[0:TB])
                q = list(carry[TB:2 * TB])
                sl = pl.ds(k * 16, 16)
                ty0 = tybuf[0, sl]
                tydv = tyd[sl]
                for i in range(TB):
                    v = (wbuf[slot, t0 + i, sl] + posbuf[slot, p0 + i, sl]
                         + (ty0 + fis[i] * tydv))
                    wbuf[slot, t0 + i, sl] = v
                    s[i] = s[i] + v
                    q[i] = q[i] + v * v
                return tuple(s) + tuple(q)

            carry = plsc.parallel_loop(
                0, HC, unroll=1, carry=(z,) * (2 * TB))(acc_body)
            ms, rs = [], []
            for i in range(TB):
                meanv = _lanesum(carry[i]) * (1.0 / H)
                varv = _lanesum(carry[TB + i]) * (1.0 / H) - meanv * meanv
                ms.append(meanv)
                rs.append(_rsqrt16(varv + 1e-12))

            def norm_body(k, t0=t0, ms=ms, rs=rs):
                sl = pl.ds(k * 16, 16)
                g = gv[sl]
                b_ = bv[sl]
                for i in range(TB):
                    v = wbuf[slot, t0 + i, sl]
                    wbuf[slot, t0 + i, sl] = (v - ms[i]) * rs[i] * g + b_

            plsc.parallel_loop(0, HC, unroll=2)(norm_body)

        for cp in out_copies(c, slot):
            cp.start()

    start_chunk(0, 0)

    def pair(cc, _):
        do_chunk(2 * cc, 0)
        do_chunk(2 * cc + 1, 1)
        return 0

    lax.fori_loop(0, NCH // 2, pair, 0)
    for j in range(2):
        for cp in out_copies(NCH - 2 + j, j):
            cp.wait()


@jax.jit
def _run(ids_flat, ttf_flat, word_emb, pos_emb, type_emb, gamma, beta):
    mesh = plsc.VectorSubcoreMesh(core_axis_name="c", subcore_axis_name="s")
    f = pl.kernel(
        _body,
        out_type=jax.ShapeDtypeStruct((B * L, H), jnp.float32),
        mesh=mesh,
        compiler_params=pltpu.CompilerParams(needs_layout_passes=False),
        scratch_types=[
            pltpu.VMEM((2 * PPW,), jnp.int32),
            pltpu.VMEM((2 * PPW + 16,), jnp.float32),
            pltpu.VMEM((2, CT, H), jnp.float32),
            pltpu.VMEM((2, CP, H), jnp.float32),
            pltpu.VMEM((T, H), jnp.float32),
            pltpu.VMEM((H,), jnp.float32),
            pltpu.VMEM((H,), jnp.float32),
            pltpu.VMEM((H,), jnp.float32),
            pltpu.SemaphoreType.DMA((2,)),
            pltpu.SemaphoreType.DMA((2,)),
            pltpu.SemaphoreType.DMA((2,)),
        ],
    )
    return f(ids_flat, ttf_flat, word_emb, pos_emb, type_emb, gamma, beta)


def kernel(input_ids, token_type_ids, word_emb, pos_emb, type_emb, gamma, beta):
    ids_flat = input_ids.reshape(-1).astype(jnp.int32)
    ttf_flat = token_type_ids.reshape(-1).astype(jnp.float32)
    out = _run(ids_flat, ttf_flat, word_emb, pos_emb, type_emb, gamma, beta)
    return out.reshape(B, L, H)


# speedup vs baseline: 1.1387x; 1.1115x over previous
import functools

import jax
import jax.numpy as jnp
from jax import lax
from jax.experimental import pallas as pl
from jax.experimental.pallas import tpu as pltpu
from jax.experimental.pallas import tpu_sc as plsc

B, L, H = 4, 4096, 1024
V, P, T = 30522, 4096, 2

NC, NS = 2, 16
NW = NC * NS
NTOK = B * L
TPW = NTOK // NW
C = 16
NCH = TPW // C
HC = H // 16


def _rsqrt16(x):
    i = plsc.bitcast(x, jnp.int32)
    i = 0x5F3759DF - lax.shift_right_logical(i, 1)
    y = plsc.bitcast(i, jnp.float32)
    xh = x * 0.5
    for _ in range(3):
        y = y * (1.5 - xh * y * y)
    return y


def _lanesum(x):
    lanes = lax.iota(jnp.int32, 16)
    for off in (8, 4, 2, 1):
        perm = lax.bitwise_and(lanes + off, 15)
        x = x + x.at[perm].get(mode="promise_in_bounds")
    return x


def _body(ids_hbm, ttf_hbm, word_hbm, pos_hbm, type_hbm, gamma_hbm, beta_hbm,
          out_hbm,
          idx_v, ttf_v, wbuf, posbuf, tybuf, tyd, gv, bv, gsem, psem, osem):
    wid = lax.axis_index("s") * NC + lax.axis_index("c")
    base = wid * TPW
    pb = lax.rem(base, L)

    pltpu.sync_copy(ids_hbm.at[pl.ds(base, TPW)], idx_v)
    pltpu.sync_copy(ttf_hbm.at[pl.ds(base, TPW)], ttf_v.at[pl.ds(0, TPW)])
    pltpu.sync_copy(type_hbm, tybuf)
    pltpu.sync_copy(gamma_hbm, gv)
    pltpu.sync_copy(beta_hbm, bv)

    def tyd_body(k):
        sl = pl.ds(k * 16, 16)
        tyd[sl] = tybuf[1, sl] - tybuf[0, sl]

    plsc.parallel_loop(0, HC, unroll=2)(tyd_body)

    def start_word(c, slot):
        pltpu.async_copy(word_hbm.at[idx_v.at[pl.ds(c * C, C)]],
                         wbuf.at[slot], gsem.at[slot])

    def start_pos(c, pslot):
        pltpu.async_copy(pos_hbm.at[pl.ds(pb + c * C, C)], posbuf.at[pslot],
                         psem.at[pslot])

    def do_chunk(c, slot, pslot):
        nslot = (slot + 2) % 4

        @pl.when(c + 2 < NCH)
        def _():
            @pl.when(c >= 2)
            def _():
                pltpu.make_async_copy(
                    wbuf.at[nslot],
                    out_hbm.at[pl.ds(base + (c - 2) * C, C)],
                    osem.at[nslot]).wait()

            start_word(c + 2, nslot)

        @pl.when(c + 1 < NCH)
        def _():
            start_pos(c + 1, 1 - pslot)

        pltpu.make_async_copy(word_hbm.at[idx_v.at[pl.ds(c * C, C)]],
                              wbuf.at[slot], gsem.at[slot]).wait()
        pltpu.make_async_copy(pos_hbm.at[pl.ds(pb + c * C, C)],
                              posbuf.at[pslot], psem.at[pslot]).wait()

        TB = 8
        z = jnp.zeros((16,), jnp.float32)
        for half in range(C // TB):
            t0 = half * TB
            ttc = ttf_v[pl.ds(c * C + t0, 16)]
            fis = [ttc.at[jnp.full((16,), i, jnp.int32)].get(
                       mode="promise_in_bounds") for i in range(TB)]

            def acc_body(k, carry, t0=t0, fis=fis):
                s = list(carry[0:TB])
                q = list(carry[TB:2 * TB])
                sl = pl.ds(k * 16, 16)
                ty0 = tybuf[0, sl]
                tydv = tyd[sl]
                for i in range(TB):
                    v = (wbuf[slot, t0 + i, sl] + posbuf[pslot, t0 + i, sl]
                         + (ty0 + fis[i] * tydv))
                    wbuf[slot, t0 + i, sl] = v
                    s[i] = s[i] + v
                    q[i] = q[i] + v * v
                return tuple(s) + tuple(q)

            carry = plsc.parallel_loop(
                0, HC, unroll=1, carry=(z,) * (2 * TB))(acc_body)
            ms, rs = [], []
            for i in range(TB):
                meanv = _lanesum(carry[i]) * (1.0 / H)
                varv = _lanesum(carry[TB + i]) * (1.0 / H) - meanv * meanv
                ms.append(meanv)
                rs.append(_rsqrt16(varv + 1e-12))

            def norm_body(k, t0=t0, ms=ms, rs=rs):
                sl = pl.ds(k * 16, 16)
                g = gv[sl]
                b = bv[sl]
                for i in range(TB):
                    v = wbuf[slot, t0 + i, sl]
                    wbuf[slot, t0 + i, sl] = (v - ms[i]) * rs[i] * g + b

            plsc.parallel_loop(0, HC, unroll=2)(norm_body)
        pltpu.async_copy(wbuf.at[slot], out_hbm.at[pl.ds(base + c * C, C)],
                         osem.at[slot])

    start_word(0, 0)
    start_pos(0, 0)
    start_word(1, 1)

    def quad(cq, _):
        for j in range(4):
            do_chunk(4 * cq + j, j, j % 2)
        return 0

    lax.fori_loop(0, NCH // 4, quad, 0)
    for j in range(4):
        pltpu.make_async_copy(wbuf.at[j],
                              out_hbm.at[pl.ds(base + (NCH - 4 + j) * C, C)],
                              osem.at[j]).wait()


@jax.jit
def _run(ids_flat, ttf_flat, word_emb, pos_emb, type_emb, gamma, beta):
    mesh = plsc.VectorSubcoreMesh(core_axis_name="c", subcore_axis_name="s")
    f = pl.kernel(
        _body,
        out_type=jax.ShapeDtypeStruct((NTOK, H), jnp.float32),
        mesh=mesh,
        compiler_params=pltpu.CompilerParams(needs_layout_passes=False),
        scratch_types=[
            pltpu.VMEM((TPW,), jnp.int32),
            pltpu.VMEM((TPW + 16,), jnp.float32),
            pltpu.VMEM((4, C, H), jnp.float32),
            pltpu.VMEM((2, C, H), jnp.float32),
            pltpu.VMEM((T, H), jnp.float32),
            pltpu.VMEM((H,), jnp.float32),
            pltpu.VMEM((H,), jnp.float32),
            pltpu.VMEM((H,), jnp.float32),
            pltpu.SemaphoreType.DMA((4,)),
            pltpu.SemaphoreType.DMA((2,)),
            pltpu.SemaphoreType.DMA((4,)),
        ],
    )
    return f(ids_flat, ttf_flat, word_emb, pos_emb, type_emb, gamma, beta)


def kernel(input_ids, token_type_ids, word_emb, pos_emb, type_emb, gamma, beta):
    ids_flat = input_ids.reshape(-1).astype(jnp.int32)
    ttf_flat = token_type_ids.reshape(-1).astype(jnp.float32)
    out = _run(ids_flat, ttf_flat, word_emb, pos_emb, type_emb, gamma, beta)
    return out.reshape(B, L, H)
